# baseline (speedup 1.0000x reference)
import functools

import jax
import jax.numpy as jnp
from jax import lax
from jax.experimental import pallas as pl
from jax.experimental.pallas import tpu as pltpu
from jax.experimental.pallas import tpu_sc as plsc

B = 1024
J = 26
C = 1000
ROW = J * C
NW = 32
ROWS_PER_W = B // NW
NBUF = 4

_mesh = plsc.VectorSubcoreMesh(core_axis_name="c", subcore_axis_name="s")


@functools.partial(
    pl.kernel,
    mesh=_mesh,
    out_type=jax.ShapeDtypeStruct((B * ROW,), jnp.int32),
    scratch_types=[
        pltpu.VMEM((ROWS_PER_W * J,), jnp.int32),
        pltpu.VMEM((NBUF * ROW,), jnp.int32),
        pltpu.SemaphoreType.DMA,
        pltpu.SemaphoreType.DMA,
        pltpu.SemaphoreType.DMA,
        pltpu.SemaphoreType.DMA,
    ],
    compiler_params=pltpu.CompilerParams(needs_layout_passes=False),
)
def _onehot_sc(x_hbm, zeros_hbm, out_hbm, xv, buf, s0, s1, s2, s3):
    sems = (s0, s1, s2, s3)
    wid = lax.axis_index("s") * 2 + lax.axis_index("c")
    base_row = wid * ROWS_PER_W
    pltpu.sync_copy(x_hbm.at[pl.ds(base_row * J, ROWS_PER_W * J)], xv)
    pltpu.sync_copy(zeros_hbm, buf)

    offs = lax.broadcasted_iota(jnp.int32, (16,), 0) * C
    ones = jnp.full((16,), 1, jnp.int32)
    zeros_v = jnp.zeros((16,), jnp.int32)

    handles = [None] * NBUF
    prev_idx = [None] * NBUF
    for k in range(ROWS_PER_W):
        b = k % NBUF
        if handles[b] is not None:
            handles[b].wait()
            pa, pb = prev_idx[b]
            plsc.store_scatter(buf, [pa], zeros_v)
            plsc.store_scatter(buf, [pb], zeros_v)
        xa = xv[pl.ds(k * J, 16)]
        xb = xv[pl.ds(k * J + (J - 16), 16)]
        ia = xa + offs + b * ROW
        ib = xb + offs + (b * ROW + (J - 16) * C)
        plsc.store_scatter(buf, [ia], ones)
        plsc.store_scatter(buf, [ib], ones)
        prev_idx[b] = (ia, ib)
        handles[b] = pltpu.async_copy(
            buf.at[pl.ds(b * ROW, ROW)],
            out_hbm.at[pl.ds((base_row + k) * ROW, ROW)],
            sems[b],
        )
    for b in range(NBUF):
        handles[b].wait()


def kernel(x):
    xf = x.reshape(-1).astype(jnp.int32)
    zeros = jnp.zeros((NBUF * ROW,), jnp.int32)
    out = _onehot_sc(xf, zeros)
    return out.reshape(B, ROW)


# speedup vs baseline: 1.9078x; 1.9078x over previous
import functools

import jax
import jax.numpy as jnp
from jax import lax
from jax.experimental import pallas as pl
from jax.experimental.pallas import tpu as pltpu
from jax.experimental.pallas import tpu_sc as plsc

B = 1024
J = 26
C = 1000
ROW = J * C
NW = 32
ROWS_PER_W = B // NW
RB = 8
NBLK = ROWS_PER_W // RB
CHUNK = 29 * 128
NCHUNK = 7
TAIL0 = NCHUNK * CHUNK
TAILW = ROW - TAIL0

_mesh = plsc.VectorSubcoreMesh(core_axis_name="c", subcore_axis_name="s")


@functools.partial(
    pl.kernel,
    mesh=_mesh,
    out_type=jax.ShapeDtypeStruct((B, ROW), jnp.int32),
    scratch_types=[
        pltpu.VMEM((ROWS_PER_W * J,), jnp.int32),
        pltpu.VMEM((RB, CHUNK), jnp.int32),
        pltpu.VMEM((RB, TAILW), jnp.int32),
    ],
    compiler_params=pltpu.CompilerParams(needs_layout_passes=False),
)
def _onehot_sc(x_hbm, zeros_hbm, out_hbm, xv, buf, tail, ):
    wid = lax.axis_index("s") * 2 + lax.axis_index("c")
    base_row = wid * ROWS_PER_W
    pltpu.sync_copy(x_hbm.at[pl.ds(base_row * J, ROWS_PER_W * J)], xv)
    pltpu.sync_copy(zeros_hbm, buf)

    offs = lax.broadcasted_iota(jnp.int32, (16,), 0) * C
    ones = jnp.full((16,), 1, jnp.int32)
    zeros_v = jnp.zeros((16,), jnp.int32)
    for r in range(RB):
        tail[r, :] = jnp.zeros((TAILW,), jnp.int32)

    for blk in range(NBLK):
        r0 = base_row + blk * RB
        cols = []
        for r in range(RB):
            k = blk * RB + r
            xa = xv[pl.ds(k * J, 16)]
            xb = xv[pl.ds(k * J + (J - 16), 16)]
            ca = xa + offs
            cb = xb + offs + (J - 16) * C
            cols.append((ca, cb))

        for r in range(RB):
            _, cb = cols[r]
            rv = jnp.full((16,), r, jnp.int32)
            m = cb >= TAIL0
            plsc.store_scatter(tail, [rv, cb - TAIL0], ones, mask=m)
        pltpu.sync_copy(tail, out_hbm.at[pl.ds(r0, RB), pl.ds(TAIL0, TAILW)])
        for r in range(RB):
            _, cb = cols[r]
            rv = jnp.full((16,), r, jnp.int32)
            m = cb >= TAIL0
            plsc.store_scatter(tail, [rv, cb - TAIL0], zeros_v, mask=m)

        for chunk in range(NCHUNK):
            c0 = chunk * CHUNK
            for r in range(RB):
                rv = jnp.full((16,), r, jnp.int32)
                for c in cols[r]:
                    m = (c >= c0) & (c < c0 + CHUNK)
                    plsc.store_scatter(buf, [rv, c - c0], ones, mask=m)
            dst = out_hbm.at[pl.ds(r0, RB), pl.ds(c0, CHUNK)]
            pltpu.sync_copy(buf, dst)
            for r in range(RB):
                rv = jnp.full((16,), r, jnp.int32)
                for c in cols[r]:
                    m = (c >= c0) & (c < c0 + CHUNK)
                    plsc.store_scatter(buf, [rv, c - c0], zeros_v, mask=m)


def kernel(x):
    xf = x.reshape(-1).astype(jnp.int32)
    zeros = jnp.zeros((RB, CHUNK), jnp.int32)
    return _onehot_sc(xf, zeros)
